# Initial kernel scaffold; baseline (speedup 1.0000x reference)
import jax
import jax.numpy as jnp
from jax.experimental import pallas as pl


def kernel(p, x, o, idx, n_o, W, bn_weight, bn_bias):
    raise NotImplementedError("write your pallas kernel here")


# speedup vs baseline: 9.0132x; 9.0132x over previous
import functools

import jax
import jax.numpy as jnp
from jax import lax
from jax.experimental import pallas as pl
from jax.experimental.pallas import tpu as pltpu
from jax.experimental.pallas import tpu_sc as plsc

B = 4
NP = 4096
C_IN = 256
C_OUT = 512
NSAMPLE = 16
N_PER = 1024
C_ALL = 3 + C_IN

_NC = 2
_NS = 16
_NW = _NC * _NS
_QPW = (B * N_PER) // _NW
_QB = 16
_CCH = C_OUT // 16


def _mm_body(a_ref, w_ref, o_ref):
    o_ref[...] = jnp.dot(a_ref[...], w_ref[...],
                         preferred_element_type=jnp.float32,
                         precision=lax.Precision.HIGHEST)


def _matmul(a, w):
    m = a.shape[0]
    tm = 512
    return pl.pallas_call(
        _mm_body,
        grid=(m // tm,),
        in_specs=[
            pl.BlockSpec((tm, C_ALL), lambda i: (i, 0)),
            pl.BlockSpec((C_ALL, C_OUT), lambda i: (0, 0)),
        ],
        out_specs=pl.BlockSpec((tm, C_OUT), lambda i: (i, 0)),
        out_shape=jax.ShapeDtypeStruct((m, C_OUT), jnp.float32),
    )(a, w)


_QT = 256


def _knn_body(q_ref, pt_ref, o_ref):
    b = pl.program_id(0)
    q = q_ref[0]
    pt = pt_ref[0]
    d0 = q[:, 0:1] - pt[0:1, :]
    d1 = q[:, 1:2] - pt[1:2, :]
    d2 = q[:, 2:3] - pt[2:3, :]
    d = d0 * d0 + d1 * d1 + d2 * d2
    iota = lax.broadcasted_iota(jnp.int32, (_QT, NP), 1)
    base = b * NP
    outs = []
    for _ in range(NSAMPLE):
        m = jnp.min(d, axis=1, keepdims=True)
        im = jnp.min(jnp.where(d <= m, iota, jnp.int32(NP)), axis=1)
        outs.append(im + base)
        d = jnp.where(iota == im[:, None], jnp.float32(3.0e38), d)
    o_ref[0] = jnp.stack(outs, axis=-1)


def _knn(qp, pt):
    return pl.pallas_call(
        _knn_body,
        grid=(B, N_PER // _QT),
        in_specs=[
            pl.BlockSpec((1, _QT, 3), lambda b, t: (b, t, 0)),
            pl.BlockSpec((1, 3, NP), lambda b, t: (b, 0, 0)),
        ],
        out_specs=pl.BlockSpec((1, _QT, NSAMPLE), lambda b, t: (b, t, 0)),
        out_shape=jax.ShapeDtypeStruct((B, N_PER, NSAMPLE), jnp.int32),
    )(qp, pt)


def _sc_body(zu_hbm, knn_hbm, gmax_hbm, gsum_hbm, gsq_hbm,
             idx_v, rows_v, mx_v, sm_v, sq_v, sem):
    wid = lax.axis_index("s") * _NC + lax.axis_index("c")
    qbase = wid * _QPW
    pltpu.sync_copy(knn_hbm.at[pl.ds(qbase, _QPW)], idx_v)

    def block_body(kb, _):
        def query_body(qb, _):
            q = kb * _QB + qb
            pltpu.async_copy(zu_hbm.at[idx_v.at[q]], rows_v, sem).wait()

            def chunk_body(cc, _):
                c0 = cc * 16
                r = rows_v[0, pl.ds(c0, 16)]
                mx = r
                sm = r
                sq = r * r
                for s in range(1, NSAMPLE):
                    r = rows_v[s, pl.ds(c0, 16)]
                    mx = jnp.maximum(mx, r)
                    sm = sm + r
                    sq = sq + r * r
                mx_v[qb, pl.ds(c0, 16)] = mx
                sm_v[qb, pl.ds(c0, 16)] = sm
                sq_v[qb, pl.ds(c0, 16)] = sq
                return 0

            lax.fori_loop(0, _CCH, chunk_body, 0)
            return 0

        lax.fori_loop(0, _QB, query_body, 0)
        row0 = qbase + kb * _QB
        pltpu.sync_copy(mx_v, gmax_hbm.at[pl.ds(row0, _QB)])
        pltpu.sync_copy(sm_v, gsum_hbm.at[pl.ds(row0, _QB)])
        pltpu.sync_copy(sq_v, gsq_hbm.at[pl.ds(row0, _QB)])
        return 0

    lax.fori_loop(0, _QPW // _QB, block_body, 0)


def _sc_gather_reduce(zu, knn):
    n = B * N_PER
    shp = jax.ShapeDtypeStruct((n, C_OUT), jnp.float32)
    mesh = plsc.VectorSubcoreMesh(core_axis_name="c", subcore_axis_name="s")
    fn = functools.partial(
        pl.kernel,
        mesh=mesh,
        out_type=[shp, shp, shp],
        scratch_types=[
            pltpu.VMEM((_QPW, NSAMPLE), jnp.int32),
            pltpu.VMEM((NSAMPLE, C_OUT), jnp.float32),
            pltpu.VMEM((_QB, C_OUT), jnp.float32),
            pltpu.VMEM((_QB, C_OUT), jnp.float32),
            pltpu.VMEM((_QB, C_OUT), jnp.float32),
            pltpu.SemaphoreType.DMA,
        ],
    )(_sc_body)
    return fn(zu, knn)


def _bn_body(gmax_ref, gsum_ref, gsq_ref, v_ref, g_ref, b_ref, o_ref):
    v = v_ref[...]
    gs = gsum_ref[...]
    total = jnp.float32(B * N_PER * NSAMPLE)
    sv = jnp.sum(v, axis=0, keepdims=True)
    ssum = jnp.sum(gs, axis=0, keepdims=True)
    ssq = jnp.sum(gsq_ref[...], axis=0, keepdims=True)
    scross = jnp.sum(v * gs, axis=0, keepdims=True)
    sv2 = jnp.sum(v * v, axis=0, keepdims=True)
    mean = (ssum - jnp.float32(NSAMPLE) * sv) / total
    esq = (ssq - 2.0 * scross + jnp.float32(NSAMPLE) * sv2) / total
    var = esq - mean * mean
    scale = g_ref[...] * lax.rsqrt(var + 1e-5)
    shift = b_ref[...] - mean * scale
    o_ref[...] = jnp.maximum((gmax_ref[...] - v) * scale + shift, 0.0)


def _bn_finalize(gmax, gsum, gsq, v, gamma, beta):
    n = B * N_PER
    return pl.pallas_call(
        _bn_body,
        out_shape=jax.ShapeDtypeStruct((n, C_OUT), jnp.float32),
    )(gmax, gsum, gsq, v, gamma.reshape(1, C_OUT), beta.reshape(1, C_OUT))


def kernel(p, x, o, idx, n_o, W, bn_weight, bn_bias):
    n = B * N_PER
    n_p = jnp.take(p, idx, axis=0)
    a_in = jnp.concatenate(
        [
            jnp.concatenate([p, x], axis=1),
            jnp.concatenate([n_p, jnp.zeros((n, C_IN), jnp.float32)], axis=1),
        ],
        axis=0,
    )
    zuv = _matmul(a_in, W)
    zu = zuv[: B * NP]
    v = zuv[B * NP:]

    knn = _knn(n_p.reshape(B, N_PER, 3),
               jnp.transpose(p.reshape(B, NP, 3), (0, 2, 1)))
    knn_flat = knn.reshape(n, NSAMPLE)

    gmax, gsum, gsq = _sc_gather_reduce(zu, knn_flat)
    x_out = _bn_finalize(gmax, gsum, gsq, v, bn_weight, bn_bias)
    return (n_p, x_out, n_o)


# speedup vs baseline: 10.9031x; 1.2097x over previous
import functools

import jax
import jax.numpy as jnp
from jax import lax
from jax.experimental import pallas as pl
from jax.experimental.pallas import tpu as pltpu
from jax.experimental.pallas import tpu_sc as plsc

B = 4
NP = 4096
C_IN = 256
C_OUT = 512
NSAMPLE = 16
N_PER = 1024
C_ALL = 3 + C_IN

_NC = 2
_NS = 16
_NW = _NC * _NS
_QPW = (B * N_PER) // _NW
_QB = 16
_CCH = C_OUT // 16


def _mm_body(a_ref, w_ref, o_ref):
    o_ref[...] = jnp.dot(a_ref[...], w_ref[...],
                         preferred_element_type=jnp.float32,
                         precision=lax.Precision.HIGHEST)


def _matmul(a, w):
    m = a.shape[0]
    tm = 512
    return pl.pallas_call(
        _mm_body,
        grid=(m // tm,),
        in_specs=[
            pl.BlockSpec((tm, C_ALL), lambda i: (i, 0)),
            pl.BlockSpec((C_ALL, C_OUT), lambda i: (0, 0)),
        ],
        out_specs=pl.BlockSpec((tm, C_OUT), lambda i: (i, 0)),
        out_shape=jax.ShapeDtypeStruct((m, C_OUT), jnp.float32),
    )(a, w)


_QT = 256


def _knn_body(q_ref, pt_ref, o_ref):
    b = pl.program_id(0)
    q = q_ref[0]
    pt = pt_ref[0]
    d0 = q[:, 0:1] - pt[0:1, :]
    d1 = q[:, 1:2] - pt[1:2, :]
    d2 = q[:, 2:3] - pt[2:3, :]
    d = d0 * d0 + d1 * d1 + d2 * d2
    iota = lax.broadcasted_iota(jnp.int32, (_QT, NP), 1)
    base = b * NP
    outs = []
    for _ in range(NSAMPLE):
        m = jnp.min(d, axis=1, keepdims=True)
        im = jnp.min(jnp.where(d <= m, iota, jnp.int32(NP)), axis=1)
        outs.append(im + base)
        d = jnp.where(iota == im[:, None], jnp.float32(3.0e38), d)
    o_ref[0] = jnp.stack(outs, axis=-1)


def _knn(qp, pt):
    return pl.pallas_call(
        _knn_body,
        grid=(B, N_PER // _QT),
        in_specs=[
            pl.BlockSpec((1, _QT, 3), lambda b, t: (b, t, 0)),
            pl.BlockSpec((1, 3, NP), lambda b, t: (b, 0, 0)),
        ],
        out_specs=pl.BlockSpec((1, _QT, NSAMPLE), lambda b, t: (b, t, 0)),
        out_shape=jax.ShapeDtypeStruct((B, N_PER, NSAMPLE), jnp.int32),
    )(qp, pt)


_GQ = 4
_GR = _GQ * NSAMPLE
_NT = _QPW // _GQ
_TPB = _QB // _GQ


def _sc_body(zu_hbm, knn_hbm, gmax_hbm, gsum_hbm, gsq_hbm,
             idx_v, rows0_v, rows1_v, mx_v, sm_v, sq_v, sem0, sem1):
    wid = lax.axis_index("s") * _NC + lax.axis_index("c")
    qbase = wid * _QPW
    pltpu.sync_copy(knn_hbm.at[wid], idx_v)

    bufs = (rows0_v, rows1_v)
    sems = (sem0, sem1)
    pltpu.async_copy(zu_hbm.at[idx_v.at[0]], rows0_v, sem0)
    pltpu.async_copy(zu_hbm.at[idx_v.at[1]], rows1_v, sem1)

    def pair_body(g, _):
        for b in range(2):
            rows_v, sem = bufs[b], sems[b]
            t = 2 * g + b
            pltpu.make_async_copy(zu_hbm.at[idx_v.at[t]], rows_v, sem).wait()
            srow0 = lax.rem(t, _TPB) * _GQ

            def chunk_body(cc, _):
                c0 = cc * 16
                for qq in range(_GQ):
                    r = rows_v[qq * NSAMPLE, pl.ds(c0, 16)]
                    mx = r
                    sm = r
                    sq = r * r
                    for s in range(1, NSAMPLE):
                        r = rows_v[qq * NSAMPLE + s, pl.ds(c0, 16)]
                        mx = jnp.maximum(mx, r)
                        sm = sm + r
                        sq = sq + r * r
                    mx_v[srow0 + qq, pl.ds(c0, 16)] = mx
                    sm_v[srow0 + qq, pl.ds(c0, 16)] = sm
                    sq_v[srow0 + qq, pl.ds(c0, 16)] = sq
                return 0

            lax.fori_loop(0, _CCH, chunk_body, 0)

            @pl.when(lax.rem(t, _TPB) == _TPB - 1)
            def _flush():
                row0 = qbase + lax.div(t, _TPB) * _QB
                pltpu.sync_copy(mx_v, gmax_hbm.at[pl.ds(row0, _QB)])
                pltpu.sync_copy(sm_v, gsum_hbm.at[pl.ds(row0, _QB)])
                pltpu.sync_copy(sq_v, gsq_hbm.at[pl.ds(row0, _QB)])

            @pl.when(t + 2 < _NT)
            def _prefetch():
                pltpu.async_copy(zu_hbm.at[idx_v.at[t + 2]], rows_v, sem)

        return 0

    lax.fori_loop(0, _NT // 2, pair_body, 0)


def _sc_gather_reduce(zu, knn):
    n = B * N_PER
    shp = jax.ShapeDtypeStruct((n, C_OUT), jnp.float32)
    mesh = plsc.VectorSubcoreMesh(core_axis_name="c", subcore_axis_name="s")
    fn = functools.partial(
        pl.kernel,
        mesh=mesh,
        out_type=[shp, shp, shp],
        scratch_types=[
            pltpu.VMEM((_NT, _GR), jnp.int32),
            pltpu.VMEM((_GR, C_OUT), jnp.float32),
            pltpu.VMEM((_GR, C_OUT), jnp.float32),
            pltpu.VMEM((_QB, C_OUT), jnp.float32),
            pltpu.VMEM((_QB, C_OUT), jnp.float32),
            pltpu.VMEM((_QB, C_OUT), jnp.float32),
            pltpu.SemaphoreType.DMA,
            pltpu.SemaphoreType.DMA,
        ],
    )(_sc_body)
    return fn(zu, knn.reshape(_NW, _NT, _GR))


def _bn_body(gmax_ref, gsum_ref, gsq_ref, v_ref, g_ref, b_ref, o_ref):
    v = v_ref[...]
    gs = gsum_ref[...]
    total = jnp.float32(B * N_PER * NSAMPLE)
    sv = jnp.sum(v, axis=0, keepdims=True)
    ssum = jnp.sum(gs, axis=0, keepdims=True)
    ssq = jnp.sum(gsq_ref[...], axis=0, keepdims=True)
    scross = jnp.sum(v * gs, axis=0, keepdims=True)
    sv2 = jnp.sum(v * v, axis=0, keepdims=True)
    mean = (ssum - jnp.float32(NSAMPLE) * sv) / total
    esq = (ssq - 2.0 * scross + jnp.float32(NSAMPLE) * sv2) / total
    var = esq - mean * mean
    scale = g_ref[...] * lax.rsqrt(var + 1e-5)
    shift = b_ref[...] - mean * scale
    o_ref[...] = jnp.maximum((gmax_ref[...] - v) * scale + shift, 0.0)


def _bn_finalize(gmax, gsum, gsq, v, gamma, beta):
    n = B * N_PER
    return pl.pallas_call(
        _bn_body,
        out_shape=jax.ShapeDtypeStruct((n, C_OUT), jnp.float32),
    )(gmax, gsum, gsq, v, gamma.reshape(1, C_OUT), beta.reshape(1, C_OUT))


def kernel(p, x, o, idx, n_o, W, bn_weight, bn_bias):
    n = B * N_PER
    n_p = jnp.take(p, idx, axis=0)
    a_in = jnp.concatenate(
        [
            jnp.concatenate([p, x], axis=1),
            jnp.concatenate([n_p, jnp.zeros((n, C_IN), jnp.float32)], axis=1),
        ],
        axis=0,
    )
    zuv = _matmul(a_in, W)
    zu = zuv[: B * NP]
    v = zuv[B * NP:]

    knn = _knn(n_p.reshape(B, N_PER, 3),
               jnp.transpose(p.reshape(B, NP, 3), (0, 2, 1)))
    knn_flat = knn.reshape(n, NSAMPLE)

    gmax, gsum, gsq = _sc_gather_reduce(zu, knn_flat)
    x_out = _bn_finalize(gmax, gsum, gsq, v, bn_weight, bn_bias)
    return (n_p, x_out, n_o)
